# Initial kernel scaffold; baseline (speedup 1.0000x reference)
import jax
import jax.numpy as jnp
from jax.experimental import pallas as pl


def kernel(input_ids, item_codes, centroids):
    raise NotImplementedError("write your pallas kernel here")


# speedup vs baseline: 64.6987x; 64.6987x over previous
import functools

import jax
import jax.numpy as jnp
from jax import lax
from jax.experimental import pallas as pl
from jax.experimental.pallas import tpu as pltpu
from jax.experimental.pallas import tpu_sc as plsc

B = 1024
L = 200
NUM_ITEMS = 100000
PQ_M = 8
EMB = 128
SUB = EMB // PQ_M
BYTES = EMB // SUB
VALS = 256

NTOK = B * L
NW = 32
TOK_PER_TILE = NTOK // NW
GTOK = 256
NGROUP = TOK_PER_TILE // GTOK
GROWS = GTOK * BYTES
IDS_ROWS_PER_TILE = TOK_PER_TILE // 128


def _sc_body(ids3d_hbm, codes_hbm, cent_hbm, out_hbm,
             ids2d_v, codes_v, idx_v, rows_v,
             sem_in, sem_g):
    wid = lax.axis_index("s") * 2 + lax.axis_index("c")

    pltpu.sync_copy(ids3d_hbm.at[wid], ids2d_v)

    iota = lax.iota(jnp.int32, 16)
    rowpat = lax.shift_right_logical(iota, 3)
    colpat = lax.bitwise_and(iota, 7)
    joff = colpat * VALS

    def group(g, _):
        c0 = pltpu.async_copy(codes_hbm.at[ids2d_v.at[2 * g]],
                              codes_v.at[pl.ds(0, 128)], sem_in)
        c1 = pltpu.async_copy(codes_hbm.at[ids2d_v.at[2 * g + 1]],
                              codes_v.at[pl.ds(128, 128)], sem_in)
        c0.wait()
        c1.wait()

        def body(i, _):
            tok_loc = 2 * i + rowpat
            tok_glob = g * GTOK + tok_loc
            ids16 = plsc.load_gather(
                ids2d_v,
                [lax.shift_right_logical(tok_glob, 7),
                 lax.bitwise_and(tok_glob, 127)])
            codes16 = plsc.load_gather(codes_v, [tok_loc, colpat])
            c = jnp.minimum(codes16, VALS - 1)
            c = jnp.where(ids16 == 0, 0, c)
            r = lax.shift_right_logical(i, 3)
            col = lax.bitwise_and(i, 7) * 16
            idx_v[r, pl.ds(col, 16)] = c + joff
            return 0

        lax.fori_loop(0, GROWS // 16, body, 0, unroll=2)

        copies = []
        for r in range(GROWS // 128):
            copies.append(pltpu.async_copy(
                cent_hbm.at[idx_v.at[r]],
                rows_v.at[pl.ds(r * 128, 128)], sem_g))
        for cp in copies:
            cp.wait()

        out_base = wid * (TOK_PER_TILE * BYTES) + g * GROWS
        pltpu.sync_copy(rows_v, out_hbm.at[pl.ds(out_base, GROWS)])
        return 0

    lax.fori_loop(0, NGROUP, group, 0)


@jax.jit
def _run(ids3d, item_codes, cent_flat):
    mesh = plsc.VectorSubcoreMesh(core_axis_name="c", subcore_axis_name="s")
    f = pl.kernel(
        _sc_body,
        out_type=jax.ShapeDtypeStruct((NTOK * BYTES, SUB), jnp.float32),
        mesh=mesh,
        compiler_params=pltpu.CompilerParams(
            needs_layout_passes=False, use_tc_tiling_on_sc=False),
        scratch_types=[
            pltpu.VMEM((IDS_ROWS_PER_TILE, 128), jnp.int32),
            pltpu.VMEM((GTOK, BYTES), jnp.int32),
            pltpu.VMEM((GROWS // 128, 128), jnp.int32),
            pltpu.VMEM((GROWS, SUB), jnp.float32),
            pltpu.SemaphoreType.DMA,
            pltpu.SemaphoreType.DMA,
        ],
    )
    return f(ids3d, item_codes, cent_flat)


def kernel(input_ids, item_codes, centroids):
    ids = input_ids.astype(jnp.int32)
    ids3d = ids.reshape(NW, IDS_ROWS_PER_TILE, 128)
    cent_flat = centroids.reshape(BYTES * VALS, SUB)
    out = _run(ids3d, item_codes.astype(jnp.int32), cent_flat)
    return out.reshape(B, L, EMB)


# speedup vs baseline: 65.5262x; 1.0128x over previous
import jax
import jax.numpy as jnp
from jax import lax
from jax.experimental import pallas as pl
from jax.experimental.pallas import tpu as pltpu
from jax.experimental.pallas import tpu_sc as plsc

B = 1024
L = 200
NUM_ITEMS = 100000
PQ_M = 8
EMB = 128
SUB = EMB // PQ_M
BYTES = EMB // SUB
VALS = 256

NTOK = B * L
NW = 32
TOK_PER_TILE = NTOK // NW
GTOK = 256
NGROUP = TOK_PER_TILE // GTOK
GROWS = GTOK * BYTES


def _sc_body(ids_hbm, codes_hbm, cent_hbm, out_hbm,
             ids_v, codes_v, idx_v, rows_v,
             sem_codes, sem_rows, sem_out):
    wid = lax.axis_index("s") * 2 + lax.axis_index("c")

    pltpu.sync_copy(ids_hbm.at[wid], ids_v)

    iota = lax.iota(jnp.int32, 16)
    rowpat = lax.shift_right_logical(iota, 3)
    colpat = lax.bitwise_and(iota, 7)
    joff = colpat * VALS

    def issue_codes(g, p):
        pltpu.async_copy(codes_hbm.at[ids_v.at[pl.ds(g * GTOK, GTOK)]],
                         codes_v.at[p], sem_codes)

    def wait_codes():
        pltpu.make_async_copy(codes_hbm.at[ids_v.at[pl.ds(0, GTOK)]],
                              codes_v.at[0], sem_codes).wait()

    def issue_rows(p):
        pltpu.async_copy(cent_hbm.at[idx_v.at[p]], rows_v.at[p], sem_rows)

    def wait_rows():
        pltpu.make_async_copy(cent_hbm.at[idx_v.at[0]], rows_v.at[0],
                              sem_rows).wait()

    def issue_out(g, p):
        base = wid * (TOK_PER_TILE * BYTES) + g * GROWS
        pltpu.async_copy(rows_v.at[p], out_hbm.at[pl.ds(base, GROWS)],
                         sem_out)

    def wait_out():
        pltpu.make_async_copy(rows_v.at[0], out_hbm.at[pl.ds(0, GROWS)],
                              sem_out).wait()

    def compute_idx(g, p):
        codes_g = codes_v.at[p]

        def body(i, _):
            tok_loc = 2 * i + rowpat
            ids16 = plsc.load_gather(ids_v, [g * GTOK + tok_loc])
            codes16 = plsc.load_gather(codes_g, [tok_loc, colpat])
            c = jnp.minimum(codes16, VALS - 1)
            c = jnp.where(ids16 == 0, 0, c)
            idx_v[p, pl.ds(16 * i, 16)] = c + joff
            return 0

        lax.fori_loop(0, GROWS // 16, body, 0, unroll=2)

    issue_codes(0, 0)

    def group(g, _):
        p = g & 1
        wait_codes()

        @pl.when(g + 1 < NGROUP)
        def _():
            issue_codes(g + 1, 1 - p)

        compute_idx(g, p)

        @pl.when(g >= 1)
        def _():
            wait_rows()
            issue_out(g - 1, 1 - p)

        @pl.when(g >= 2)
        def _():
            wait_out()

        issue_rows(p)
        return 0

    lax.fori_loop(0, NGROUP, group, 0)

    wait_rows()
    issue_out(NGROUP - 1, (NGROUP - 1) & 1)
    wait_out()
    wait_out()


@jax.jit
def _run(ids2d, item_codes, cent_flat):
    mesh = plsc.VectorSubcoreMesh(core_axis_name="c", subcore_axis_name="s")
    f = pl.kernel(
        _sc_body,
        out_type=jax.ShapeDtypeStruct((NTOK * BYTES, SUB), jnp.float32),
        mesh=mesh,
        compiler_params=pltpu.CompilerParams(
            needs_layout_passes=False, use_tc_tiling_on_sc=False),
        scratch_types=[
            pltpu.VMEM((TOK_PER_TILE,), jnp.int32),
            pltpu.VMEM((2, GTOK, BYTES), jnp.int32),
            pltpu.VMEM((2, GROWS), jnp.int32),
            pltpu.VMEM((2, GROWS, SUB), jnp.float32),
            pltpu.SemaphoreType.DMA,
            pltpu.SemaphoreType.DMA,
            pltpu.SemaphoreType.DMA,
        ],
    )
    return f(ids2d, item_codes, cent_flat)


def kernel(input_ids, item_codes, centroids):
    ids = input_ids.astype(jnp.int32)
    ids2d = ids.reshape(NW, TOK_PER_TILE)
    cent_flat = centroids.reshape(BYTES * VALS, SUB)
    out = _run(ids2d, item_codes.astype(jnp.int32), cent_flat)
    return out.reshape(B, L, EMB)


# speedup vs baseline: 147.0707x; 2.2445x over previous
import jax
import jax.numpy as jnp
from jax import lax
from jax.experimental import pallas as pl
from jax.experimental.pallas import tpu as pltpu
from jax.experimental.pallas import tpu_sc as plsc

B = 1024
L = 200
NUM_ITEMS = 100000
PQ_M = 8
EMB = 128
SUB = EMB // PQ_M
BYTES = EMB // SUB
VALS = 256

NTOK = B * L
NW = 32
TOK_PER_TILE = NTOK // NW
GTOK = 256
NGROUP = TOK_PER_TILE // GTOK
GROWS = GTOK * BYTES


def _sc_body(ids_hbm, codes_hbm, cent_hbm, out_hbm,
             ids_v, codes_v, idx_v, rows_v, cent_sp,
             sem_codes, sem_rows, sem_out):
    wid = lax.axis_index("s") * 2 + lax.axis_index("c")

    @pl.when(lax.axis_index("s") == 0)
    def _():
        pltpu.sync_copy(cent_hbm, cent_sp)

    pltpu.sync_copy(ids_hbm.at[wid], ids_v)
    plsc.subcore_barrier()

    iota = lax.iota(jnp.int32, 16)
    rowpat = lax.shift_right_logical(iota, 3)
    colpat = lax.bitwise_and(iota, 7)
    joff = colpat * VALS

    def issue_codes(g, p):
        pltpu.async_copy(codes_hbm.at[ids_v.at[pl.ds(g * GTOK, GTOK)]],
                         codes_v.at[p], sem_codes)

    def wait_codes():
        pltpu.make_async_copy(codes_hbm.at[ids_v.at[pl.ds(0, GTOK)]],
                              codes_v.at[0], sem_codes).wait()

    def issue_rows(p):
        pltpu.async_copy(cent_sp.at[idx_v.at[p]], rows_v.at[p], sem_rows)

    def wait_rows():
        pltpu.make_async_copy(cent_sp.at[idx_v.at[0]], rows_v.at[0],
                              sem_rows).wait()

    def issue_out(g, p):
        base = wid * (TOK_PER_TILE * BYTES) + g * GROWS
        pltpu.async_copy(rows_v.at[p], out_hbm.at[pl.ds(base, GROWS)],
                         sem_out)

    def wait_out():
        pltpu.make_async_copy(rows_v.at[0], out_hbm.at[pl.ds(0, GROWS)],
                              sem_out).wait()

    def compute_idx(g, p):
        codes_g = codes_v.at[p]

        def body(i, _):
            tok_loc = 2 * i + rowpat
            ids16 = plsc.load_gather(ids_v, [g * GTOK + tok_loc])
            codes16 = plsc.load_gather(codes_g, [tok_loc, colpat])
            c = jnp.where(ids16 == 0, 0, codes16)
            idx_v[p, pl.ds(16 * i, 16)] = c + joff
            return 0

        lax.fori_loop(0, GROWS // 16, body, 0, unroll=4)

    issue_codes(0, 0)

    def group(g, _):
        p = g & 1
        wait_codes()

        @pl.when(g + 1 < NGROUP)
        def _():
            issue_codes(g + 1, 1 - p)

        compute_idx(g, p)

        @pl.when(g >= 1)
        def _():
            wait_rows()
            issue_out(g - 1, 1 - p)

        @pl.when(g >= 2)
        def _():
            wait_out()

        issue_rows(p)
        return 0

    lax.fori_loop(0, NGROUP, group, 0)

    wait_rows()
    issue_out(NGROUP - 1, (NGROUP - 1) & 1)
    wait_out()
    wait_out()


@jax.jit
def _run(ids2d, item_codes, cent_flat):
    mesh = plsc.VectorSubcoreMesh(core_axis_name="c", subcore_axis_name="s")
    f = pl.kernel(
        _sc_body,
        out_type=jax.ShapeDtypeStruct((NTOK * BYTES, SUB), jnp.float32),
        mesh=mesh,
        compiler_params=pltpu.CompilerParams(
            needs_layout_passes=False, use_tc_tiling_on_sc=False),
        scratch_types=[
            pltpu.VMEM((TOK_PER_TILE,), jnp.int32),
            pltpu.VMEM((2, GTOK, BYTES), jnp.int32),
            pltpu.VMEM((2, GROWS), jnp.int32),
            pltpu.VMEM((2, GROWS, SUB), jnp.float32),
            pltpu.VMEM_SHARED((BYTES * VALS, SUB), jnp.float32),
            pltpu.SemaphoreType.DMA,
            pltpu.SemaphoreType.DMA,
            pltpu.SemaphoreType.DMA,
        ],
    )
    return f(ids2d, item_codes, cent_flat)


def kernel(input_ids, item_codes, centroids):
    ids = input_ids.astype(jnp.int32)
    ids2d = ids.reshape(NW, TOK_PER_TILE)
    cent_flat = centroids.reshape(BYTES * VALS, SUB)
    out = _run(ids2d, item_codes.astype(jnp.int32), cent_flat)
    return out.reshape(B, L, EMB)
